# baseline (speedup 1.0000x reference)
import functools

import jax
import jax.numpy as jnp
from jax import lax
from jax.experimental import pallas as pl
from jax.experimental.pallas import tpu as pltpu
from jax.experimental.pallas import tpu_sc as plsc

_V, _P, _D, _NCAP, _NM = 50000, 100000, 128, 4096, 320000
_TW = 144
_NC, _NS = 2, 16
_NW = _NC * _NS
_K = 125
_RW = _NM // (_NW * _K)
_NROWS = _NM // _K
_NBUF = 3
_VBLK = 2000


def _tables_body(scale_ref, f_ref, c_ref, tab_ref):
    x = f_ref[...]
    ssq = jnp.sum(x * x, axis=1, keepdims=True)
    inv = 1.0 / jnp.maximum(jnp.sqrt(ssq), 1e-12)
    fn = x * inv
    logits = lax.dot_general(fn, c_ref[...], (((1,), (1,)), ((), ())),
                             preferred_element_type=jnp.float32) * scale_ref[0]
    m = jnp.max(logits, axis=1, keepdims=True)
    lse = m + jnp.log(jnp.sum(jnp.exp(logits - m), axis=1, keepdims=True))
    col = lax.broadcasted_iota(jnp.int32, (_VBLK, _TW - _D), 1)
    extras = jnp.where(col == 0, jnp.float32(1.0), jnp.float32(0.0))
    tab_ref[...] = jnp.concatenate([logits - lse, extras], axis=1)


def _build_table(scale, features, caption_embed):
    return pl.pallas_call(
        _tables_body,
        grid=(_V // _VBLK,),
        in_specs=[
            pl.BlockSpec(memory_space=pltpu.SMEM),
            pl.BlockSpec((_VBLK, _D), lambda i: (i, 0)),
            pl.BlockSpec((_D, _D), lambda i: (0, 0)),
        ],
        out_specs=pl.BlockSpec((_VBLK, _TW), lambda i: (i, 0)),
        out_shape=jax.ShapeDtypeStruct((_V, _TW), jnp.float32),
    )(scale, features, caption_embed)


def _sc_body(tab_hbm, v2p_hbm, c2p_hbm, seg_hbm, z_hbm,
             accout,
             c2p_v, seg_v, idx2_v, st, acc_sh,
             sem_i, sem_g, sem_s):
    c = lax.axis_index("c")
    s = lax.axis_index("s")
    w = s * _NC + c
    rb = w * _RW
    zrows = _NCAP // _NS

    pltpu.sync_copy(z_hbm.at[pl.ds(s * zrows, zrows)],
                    acc_sh.at[pl.ds(s * zrows, zrows)])

    pltpu.sync_copy(seg_hbm.at[pl.ds(rb, _RW)], seg_v)
    pltpu.sync_copy(c2p_hbm.at[pl.ds(rb, _RW)], c2p_v)

    def _fire_idx(j, carry):
        pltpu.async_copy(v2p_hbm.at[c2p_v.at[j]], idx2_v.at[j], sem_i)
        return carry
    lax.fori_loop(0, _RW, _fire_idx, 0)

    def _drain_idx(j, carry):
        pltpu.make_async_copy(v2p_hbm.at[c2p_v.at[j]], idx2_v.at[j],
                              sem_i).wait()
        return carry
    lax.fori_loop(0, _RW, _drain_idx, 0)

    plsc.subcore_barrier()

    def _gather(j):
        pltpu.async_copy(tab_hbm.at[idx2_v.at[j]],
                         st.at[lax.rem(j, _NBUF)], sem_g)

    def _wait_gather(j):
        pltpu.make_async_copy(tab_hbm.at[idx2_v.at[j]],
                              st.at[lax.rem(j, _NBUF)], sem_g).wait()

    def _scatter(j):
        pltpu.async_copy(st.at[lax.rem(j, _NBUF)],
                         acc_sh.at[seg_v.at[j]], sem_s, add=True)

    def _wait_scatter(j):
        pltpu.make_async_copy(st.at[lax.rem(j, _NBUF)],
                              acc_sh.at[seg_v.at[j]], sem_s).wait()

    _gather(0)
    _gather(1)

    def _step(j, carry):
        _wait_gather(j)

        @pl.when(j >= 1)
        def _():
            _wait_scatter(j - 1)
        _scatter(j)

        @pl.when(j + 2 < _RW)
        def _():
            _gather(j + 2)
        return carry
    lax.fori_loop(0, _RW, _step, 0)

    _wait_scatter(_RW - 1)

    plsc.subcore_barrier()

    pltpu.sync_copy(acc_sh.at[pl.ds(s * zrows, zrows)],
                    accout.at[c, pl.ds(s * zrows, zrows)])


def _segment_accumulate(score_tab, v2p_map, c2p_rows, seg_rows, zeros_blk):
    mesh = plsc.VectorSubcoreMesh(core_axis_name="c", subcore_axis_name="s",
                                  num_cores=_NC, num_subcores=_NS)
    run = pl.kernel(
        _sc_body,
        compiler_params=pltpu.CompilerParams(use_tc_tiling_on_sc=False),
        out_type=jax.ShapeDtypeStruct((_NC, _NCAP, _TW), jnp.float32),
        mesh=mesh,
        scratch_types=[
            pltpu.VMEM((_RW, _K), jnp.int32),
            pltpu.VMEM((_RW, _K), jnp.int32),
            pltpu.VMEM((_RW, _K), jnp.int32),
            pltpu.VMEM((_NBUF, _K, _TW), jnp.float32),
            pltpu.VMEM_SHARED((_NCAP, _TW), jnp.float32),
            pltpu.SemaphoreType.DMA,
            pltpu.SemaphoreType.DMA,
            pltpu.SemaphoreType.DMA,
        ],
    )
    return run(score_tab, v2p_map, c2p_rows, seg_rows, zeros_blk)


def _finish_body(acc_ref, cidx_ref, pooled_ref, rn_ref, lab_ref):
    a = acc_ref[0] + acc_ref[1]
    cnt = a[:, _D:_D + 1]
    has = cnt > 0
    invc = jnp.where(has, 1.0 / jnp.where(has, cnt, 1.0), 0.0)
    pooled_ref[...] = a[:, :_D] * invc
    rn_ref[...] = cnt
    lab_ref[...] = jnp.where(has, cidx_ref[...], -100)


def _finish(acc, caption_idx):
    return pl.pallas_call(
        _finish_body,
        in_specs=[
            pl.BlockSpec((_NC, _NCAP, _TW), lambda: (0, 0, 0)),
            pl.BlockSpec((_NCAP, 1), lambda: (0, 0)),
        ],
        out_specs=[
            pl.BlockSpec((_NCAP, _D), lambda: (0, 0)),
            pl.BlockSpec((_NCAP, 1), lambda: (0, 0)),
            pl.BlockSpec((_NCAP, 1), lambda: (0, 0)),
        ],
        out_shape=[
            jax.ShapeDtypeStruct((_NCAP, _D), jnp.float32),
            jax.ShapeDtypeStruct((_NCAP, 1), jnp.float32),
            jax.ShapeDtypeStruct((_NCAP, 1), jnp.int32),
        ],
    )(acc, caption_idx)


def kernel(features, v2p_map, caption_embed, logit_scale, c2p_flat,
           caption_seg, origin_idx, caption_idx):
    scale = jnp.exp(logit_scale).astype(jnp.float32).reshape((1,))
    score_tab = _build_table(scale, features, caption_embed.astype(jnp.float32))

    c2p_rows = c2p_flat.astype(jnp.int32).reshape((_NROWS, _K))
    seg_rows = caption_seg.astype(jnp.int32).reshape((_NROWS, _K))
    zeros_blk = jnp.zeros((_NCAP, _TW), jnp.float32)
    acc = _segment_accumulate(score_tab, v2p_map.astype(jnp.int32),
                              c2p_rows, seg_rows, zeros_blk)

    pooled, rn, lab = _finish(acc,
                              caption_idx.astype(jnp.int32).reshape(
                                  (_NCAP, 1)))
    return (pooled, rn.reshape((_NCAP,)), lab.reshape((_NCAP,)),
            jnp.zeros((), jnp.float32))


# speedup vs baseline: 1.4225x; 1.4225x over previous
import functools

import jax
import jax.numpy as jnp
from jax import lax
from jax.experimental import pallas as pl
from jax.experimental.pallas import tpu as pltpu
from jax.experimental.pallas import tpu_sc as plsc

_V, _P, _D, _NCAP, _NM = 50000, 100000, 128, 4096, 320000
_CW = 16
_NC, _NS = 2, 16
_NW = _NC * _NS
_K = 125
_RW = _NM // (_NW * _K)
_NROWS = _NM // _K
_NBUF = 3
_IW = 4
_VBLK = 2000


def _tables_body(scale_ref, f_ref, c_ref, tab_ref):
    x = f_ref[...]
    ssq = jnp.sum(x * x, axis=1, keepdims=True)
    inv = 1.0 / jnp.maximum(jnp.sqrt(ssq), 1e-12)
    fn = x * inv
    logits = lax.dot_general(fn, c_ref[...], (((1,), (1,)), ((), ())),
                             preferred_element_type=jnp.float32) * scale_ref[0]
    m = jnp.max(logits, axis=1, keepdims=True)
    lse = m + jnp.log(jnp.sum(jnp.exp(logits - m), axis=1, keepdims=True))
    tab_ref[...] = logits - lse


def _build_table(scale, features, caption_embed):
    return pl.pallas_call(
        _tables_body,
        grid=(_V // _VBLK,),
        in_specs=[
            pl.BlockSpec(memory_space=pltpu.SMEM),
            pl.BlockSpec((_VBLK, _D), lambda i: (i, 0)),
            pl.BlockSpec((_D, _D), lambda i: (0, 0)),
        ],
        out_specs=pl.BlockSpec((_VBLK, _D), lambda i: (i, 0)),
        out_shape=jax.ShapeDtypeStruct((_V, _D), jnp.float32),
    )(scale, features, caption_embed)


def _sc_body(tab_hbm, v2p_hbm, c2p_hbm, seg_hbm, zfn_hbm, zcnt_hbm, ones_hbm,
             accout, cntout,
             c2p_v, seg_v, idx2_v, st, ones_v, acc_sh, cnt_sh,
             sem_ia, sem_ib, sem_ic, sem_id, sem_g, sem_s, sem_c):
    sem_iw = (sem_ia, sem_ib, sem_ic, sem_id)
    c = lax.axis_index("c")
    s = lax.axis_index("s")
    w = s * _NC + c
    rb = w * _RW
    zrows = _NCAP // _NS

    pltpu.sync_copy(zfn_hbm.at[pl.ds(s * zrows, zrows)],
                    acc_sh.at[pl.ds(s * zrows, zrows)])
    pltpu.sync_copy(zcnt_hbm.at[pl.ds(s * zrows, zrows)],
                    cnt_sh.at[pl.ds(s * zrows, zrows)])

    pltpu.sync_copy(seg_hbm.at[pl.ds(rb, _RW)], seg_v)
    pltpu.sync_copy(ones_hbm, ones_v)

    pltpu.sync_copy(c2p_hbm.at[pl.ds(rb, _RW)], c2p_v)

    def _idx_desc(b, sem):
        return pltpu.make_async_copy(v2p_hbm.at[c2p_v.at[b]],
                                     idx2_v.at[b], sem)

    def _fire_idx(b):
        for k in range(_IW):
            @pl.when(lax.rem(b, _IW) == k)
            def _(sem=sem_iw[k]):
                _idx_desc(b, sem).start()

    def _wait_idx_and_refill(b):
        for k in range(_IW):
            @pl.when(lax.rem(b, _IW) == k)
            def _(sem=sem_iw[k]):
                _idx_desc(b, sem).wait()

        @pl.when(b + _IW < _RW)
        def _():
            _fire_idx(b + _IW)

    for k in range(_IW):
        _fire_idx(k)

    plsc.subcore_barrier()

    def _gather(j):
        _wait_idx_and_refill(j)
        pltpu.async_copy(tab_hbm.at[idx2_v.at[j]],
                         st.at[lax.rem(j, _NBUF)], sem_g)

    def _wait_gather(j):
        pltpu.make_async_copy(tab_hbm.at[idx2_v.at[j]],
                              st.at[lax.rem(j, _NBUF)], sem_g).wait()

    def _scatter(j):
        pltpu.async_copy(st.at[lax.rem(j, _NBUF)],
                         acc_sh.at[seg_v.at[j]], sem_s, add=True)
        pltpu.async_copy(ones_v, cnt_sh.at[seg_v.at[j]], sem_c, add=True)

    def _wait_scatter(j):
        pltpu.make_async_copy(st.at[lax.rem(j, _NBUF)],
                              acc_sh.at[seg_v.at[j]], sem_s).wait()

    _gather(0)
    _gather(1)

    def _step(j, carry):
        _wait_gather(j)

        @pl.when(j >= 1)
        def _():
            _wait_scatter(j - 1)
        _scatter(j)

        @pl.when(j + 2 < _RW)
        def _():
            _gather(j + 2)
        return carry
    lax.fori_loop(0, _RW, _step, 0)

    _wait_scatter(_RW - 1)

    def _drain_cnt(j, carry):
        pltpu.make_async_copy(ones_v, cnt_sh.at[seg_v.at[j]], sem_c).wait()
        return carry
    lax.fori_loop(0, _RW, _drain_cnt, 0)

    plsc.subcore_barrier()

    pltpu.sync_copy(acc_sh.at[pl.ds(s * zrows, zrows)],
                    accout.at[c, pl.ds(s * zrows, zrows)])
    pltpu.sync_copy(cnt_sh.at[pl.ds(s * zrows, zrows)],
                    cntout.at[c, pl.ds(s * zrows, zrows)])


def _segment_accumulate(score_tab, v2p_map, c2p_rows, seg_rows,
                        zfn, zcnt, ones_blk):
    mesh = plsc.VectorSubcoreMesh(core_axis_name="c", subcore_axis_name="s",
                                  num_cores=_NC, num_subcores=_NS)
    run = pl.kernel(
        _sc_body,
        compiler_params=pltpu.CompilerParams(use_tc_tiling_on_sc=False),
        out_type=[
            jax.ShapeDtypeStruct((_NC, _NCAP, _D), jnp.float32),
            jax.ShapeDtypeStruct((_NC, _NCAP, _CW), jnp.float32),
        ],
        mesh=mesh,
        scratch_types=[
            pltpu.VMEM((_RW, _K), jnp.int32),
            pltpu.VMEM((_RW, _K), jnp.int32),
            pltpu.VMEM((_RW, _K), jnp.int32),
            pltpu.VMEM((_NBUF, _K, _D), jnp.float32),
            pltpu.VMEM((_K, _CW), jnp.float32),
            pltpu.VMEM_SHARED((_NCAP, _D), jnp.float32),
            pltpu.VMEM_SHARED((_NCAP, _CW), jnp.float32),
        ] + [pltpu.SemaphoreType.DMA] * 7,
    )
    return run(score_tab, v2p_map, c2p_rows, seg_rows, zfn, zcnt, ones_blk)


def _finish_body(acc_ref, cnt_ref, cidx_ref, pooled_ref, rn_ref, lab_ref):
    a = acc_ref[0] + acc_ref[1]
    cnt = cnt_ref[0, :, 0:1] + cnt_ref[1, :, 0:1]
    has = cnt > 0
    invc = jnp.where(has, 1.0 / jnp.where(has, cnt, 1.0), 0.0)
    pooled_ref[...] = a * invc
    rn_ref[...] = cnt
    lab_ref[...] = jnp.where(has, cidx_ref[...], -100)


def _finish(acc, cntacc, caption_idx):
    return pl.pallas_call(
        _finish_body,
        in_specs=[
            pl.BlockSpec((_NC, _NCAP, _D), lambda: (0, 0, 0)),
            pl.BlockSpec((_NC, _NCAP, _CW), lambda: (0, 0, 0)),
            pl.BlockSpec((_NCAP, 1), lambda: (0, 0)),
        ],
        out_specs=[
            pl.BlockSpec((_NCAP, _D), lambda: (0, 0)),
            pl.BlockSpec((_NCAP, 1), lambda: (0, 0)),
            pl.BlockSpec((_NCAP, 1), lambda: (0, 0)),
        ],
        out_shape=[
            jax.ShapeDtypeStruct((_NCAP, _D), jnp.float32),
            jax.ShapeDtypeStruct((_NCAP, 1), jnp.float32),
            jax.ShapeDtypeStruct((_NCAP, 1), jnp.int32),
        ],
    )(acc, cntacc, caption_idx)


def kernel(features, v2p_map, caption_embed, logit_scale, c2p_flat,
           caption_seg, origin_idx, caption_idx):
    scale = jnp.exp(logit_scale).astype(jnp.float32).reshape((1,))
    score_tab = _build_table(scale, features, caption_embed.astype(jnp.float32))

    c2p_rows = c2p_flat.astype(jnp.int32).reshape((_NROWS, _K))
    seg_rows = caption_seg.astype(jnp.int32).reshape((_NROWS, _K))
    zfn = jnp.zeros((_NCAP, _D), jnp.float32)
    zcnt = jnp.zeros((_NCAP, _CW), jnp.float32)
    ones_blk = jnp.ones((_K, _CW), jnp.float32)
    acc, cntacc = _segment_accumulate(score_tab, v2p_map.astype(jnp.int32),
                                      c2p_rows, seg_rows, zfn, zcnt, ones_blk)

    pooled, rn, lab = _finish(acc, cntacc,
                              caption_idx.astype(jnp.int32).reshape(
                                  (_NCAP, 1)))
    return (pooled, rn.reshape((_NCAP,)), lab.reshape((_NCAP,)),
            jnp.zeros((), jnp.float32))
